# Initial kernel scaffold; baseline (speedup 1.0000x reference)
import jax
import jax.numpy as jnp
from jax.experimental import pallas as pl


def kernel(z, idx_i, idx_j, r_ij, embed_table):
    raise NotImplementedError("write your pallas kernel here")


# speedup vs baseline: 5.4117x; 5.4117x over previous
import functools

import jax
import jax.numpy as jnp
from jax import lax
from jax.experimental import pallas as pl
from jax.experimental.pallas import tpu as pltpu
from jax.experimental.pallas import tpu_sc as plsc

_N_RBF = 8
_R_CUT = 5.0
_C0 = 0.28209479177387814
_C1 = 0.4886025119029199

_LANES = 16
_SC_CHUNK = 2000


def _sc_gather_w(z, idx_i, idx_j, table_pad, n_nodes, n_edges):
    info = plsc.get_sparse_core_info()
    nc, ns = info.num_cores, info.num_subcores
    nw = nc * ns
    per_tile = n_edges // nw
    chunk = _SC_CHUNK
    n_chunks = per_tile // chunk
    mesh = plsc.VectorSubcoreMesh(core_axis_name="c", subcore_axis_name="s")

    @functools.partial(
        pl.kernel,
        mesh=mesh,
        out_type=jax.ShapeDtypeStruct((n_edges,), jnp.float32),
        scratch_types=[
            pltpu.VMEM((n_nodes,), jnp.int32),
            pltpu.VMEM((table_pad.shape[0],), jnp.float32),
            pltpu.VMEM((chunk,), jnp.int32),
            pltpu.VMEM((chunk,), jnp.int32),
            pltpu.VMEM((chunk,), jnp.float32),
        ],
        compiler_params=pltpu.CompilerParams(
            needs_layout_passes=False,
            use_tc_tiling_on_sc=False,
        ),
    )
    def body(z_hbm, ii_hbm, jj_hbm, tab_hbm, w_hbm, z_v, tab_v, ii_v, jj_v, w_v):
        wid = lax.axis_index("s") * nc + lax.axis_index("c")
        base = wid * per_tile
        pltpu.sync_copy(z_hbm, z_v)
        pltpu.sync_copy(tab_hbm, tab_v)

        def vec_body(t, _):
            ii = ii_v[pl.ds(t * _LANES, _LANES)]
            jj = jj_v[pl.ds(t * _LANES, _LANES)]
            zi = plsc.load_gather(z_v, [ii])
            zj = plsc.load_gather(z_v, [jj])
            xi = plsc.load_gather(tab_v, [zi])
            xj = plsc.load_gather(tab_v, [zj])
            w_v[pl.ds(t * _LANES, _LANES)] = xi * xj
            return _

        for c in range(n_chunks):
            off = base + c * chunk
            pltpu.sync_copy(ii_hbm.at[pl.ds(off, chunk)], ii_v)
            pltpu.sync_copy(jj_hbm.at[pl.ds(off, chunk)], jj_v)
            lax.fori_loop(0, chunk // _LANES, vec_body, 0)
            pltpu.sync_copy(w_v, w_hbm.at[pl.ds(off, chunk)])

    return body(z, idx_i, idx_j, table_pad)


def _tc_body(r_ref, w_ref, out_ref):
    r = r_ref[...]
    w = w_ref[...]
    d2 = jnp.sum(r * r, axis=1, keepdims=True)
    inv_d = lax.rsqrt(d2)
    d = d2 * inv_d
    b = out_ref.shape[0]
    k = lax.broadcasted_iota(jnp.int32, (b, 32), 1)
    n = (k % _N_RBF + 1).astype(jnp.float32)
    s = jnp.sin(d * (jnp.pi / _R_CUT) * n)
    m = k // _N_RBF
    comp = jnp.where(m == 1, r[:, 1:2], jnp.where(m == 2, r[:, 2:3], r[:, 0:1]))
    sph = jnp.where(m == 0, _C0, _C1 * inv_d * comp)
    scale = w * (jnp.sqrt(2.0 / _R_CUT) * inv_d)
    out_ref[...] = s * sph * scale


def kernel(z, idx_i, idx_j, r_ij, embed_table):
    n_edges = idx_i.shape[0]
    n_nodes = z.shape[0]
    table_flat = embed_table.reshape(-1)
    pad = (-table_flat.shape[0]) % 128
    table_pad = jnp.pad(table_flat, (0, pad))

    w = _sc_gather_w(z.astype(jnp.int32), idx_i.astype(jnp.int32),
                     idx_j.astype(jnp.int32), table_pad, n_nodes, n_edges)

    be = 4000
    grid = n_edges // be
    out = pl.pallas_call(
        _tc_body,
        grid=(grid,),
        in_specs=[
            pl.BlockSpec((be, 3), lambda i: (i, 0)),
            pl.BlockSpec((be, 1), lambda i: (i, 0)),
        ],
        out_specs=pl.BlockSpec((be, 32), lambda i: (i, 0)),
        out_shape=jax.ShapeDtypeStruct((n_edges, 32), jnp.float32),
        compiler_params=pltpu.CompilerParams(
            dimension_semantics=("arbitrary",),
        ),
    )(r_ij, w.reshape(n_edges, 1))
    return out


# speedup vs baseline: 5.8556x; 1.0820x over previous
import functools

import jax
import jax.numpy as jnp
from jax import lax
from jax.experimental import pallas as pl
from jax.experimental.pallas import tpu as pltpu
from jax.experimental.pallas import tpu_sc as plsc

_N_RBF = 8
_R_CUT = 5.0
_C0 = 0.28209479177387814
_C1 = 0.4886025119029199

_LANES = 16
_SC_CHUNK = 2000


def _sc_gather_w(z, idx_i, idx_j, table_pad, n_nodes, n_edges, n_out):
    info = plsc.get_sparse_core_info()
    nc, ns = info.num_cores, info.num_subcores
    nw = nc * ns
    per_tile = n_edges // nw
    chunk = _SC_CHUNK
    n_chunks = per_tile // chunk
    mesh = plsc.VectorSubcoreMesh(core_axis_name="c", subcore_axis_name="s")

    @functools.partial(
        pl.kernel,
        mesh=mesh,
        out_type=jax.ShapeDtypeStruct((n_out,), jnp.float32),
        scratch_types=[
            pltpu.VMEM((n_nodes,), jnp.int32),
            pltpu.VMEM((table_pad.shape[0],), jnp.float32),
            pltpu.VMEM((chunk,), jnp.int32),
            pltpu.VMEM((chunk,), jnp.int32),
            pltpu.VMEM((chunk,), jnp.float32),
        ],
        compiler_params=pltpu.CompilerParams(
            needs_layout_passes=False,
            use_tc_tiling_on_sc=False,
        ),
    )
    def body(z_hbm, ii_hbm, jj_hbm, tab_hbm, w_hbm, z_v, tab_v, ii_v, jj_v, w_v):
        wid = lax.axis_index("s") * nc + lax.axis_index("c")
        base = wid * per_tile
        pltpu.sync_copy(z_hbm, z_v)
        pltpu.sync_copy(tab_hbm, tab_v)

        def vec_body(t, _):
            ii = ii_v[pl.ds(t * _LANES, _LANES)]
            jj = jj_v[pl.ds(t * _LANES, _LANES)]
            zi = plsc.load_gather(z_v, [ii])
            zj = plsc.load_gather(z_v, [jj])
            xi = plsc.load_gather(tab_v, [zi])
            xj = plsc.load_gather(tab_v, [zj])
            w_v[pl.ds(t * _LANES, _LANES)] = xi * xj
            return _

        for c in range(n_chunks):
            off = base + c * chunk
            pltpu.sync_copy(ii_hbm.at[pl.ds(off, chunk)], ii_v)
            pltpu.sync_copy(jj_hbm.at[pl.ds(off, chunk)], jj_v)
            lax.fori_loop(0, chunk // _LANES, vec_body, 0)
            pltpu.sync_copy(w_v, w_hbm.at[pl.ds(off, chunk)])

    return body(z, idx_i, idx_j, table_pad)


def _tc_dense_body(rx_ref, ry_ref, rz_ref, w_ref, *out_refs):
    rx = rx_ref[...]
    ry = ry_ref[...]
    rz = rz_ref[...]
    w = w_ref[...]
    d2 = rx * rx + ry * ry + rz * rz
    inv_d = lax.rsqrt(d2)
    x = d2 * inv_d * (jnp.pi / _R_CUT)
    s1 = jnp.sin(x)
    c = jnp.cos(x)
    two_c = 2.0 * c
    scale = w * (jnp.sqrt(2.0 / _R_CUT)) * inv_d
    s_prev, s_cur = jnp.zeros_like(s1), s1
    for n in range(_N_RBF):
        out_refs[n][...] = scale * s_cur
        s_prev, s_cur = s_cur, two_c * s_cur - s_prev
    out_refs[8][...] = (_C1 * ry) * inv_d
    out_refs[9][...] = (_C1 * rz) * inv_d
    out_refs[10][...] = (_C1 * rx) * inv_d


def _tc_expand_body(*refs):
    in_refs, out_ref = refs[:11], refs[11]
    bq = out_ref.shape[0]
    kj8 = lax.broadcasted_iota(jnp.int32, (32, 128), 0)
    kl8 = lax.broadcasted_iota(jnp.int32, (32, 128), 1)
    s8 = ((kl8 // 32 == kj8 % 4) & (kl8 % 8 == kj8 // 4)).astype(jnp.float32)
    kj3 = lax.broadcasted_iota(jnp.int32, (12, 128), 0)
    kl3 = lax.broadcasted_iota(jnp.int32, (12, 128), 1)
    s3 = ((kl3 // 32 == kj3 % 4)
          & ((kl3 % 32) // 8 == kj3 // 4 + 1)).astype(jnp.float32)
    cat8 = jnp.concatenate([in_refs[n][...] for n in range(8)], axis=1)
    cat3 = jnp.concatenate([in_refs[n][...] for n in range(8, 11)], axis=1)
    dn = (((1,), (0,)), ((), ()))
    rbf = lax.dot_general(cat8, s8, dn, preferred_element_type=jnp.float32)
    comp = lax.dot_general(cat3, s3, dn, preferred_element_type=jnp.float32)
    kl = lax.broadcasted_iota(jnp.int32, (bq, 128), 1)
    sph = jnp.where((kl % 32) // 8 == 0, _C0, comp)
    out_ref[...] = rbf * sph


def kernel(z, idx_i, idx_j, r_ij, embed_table):
    n_edges = idx_i.shape[0]
    n_nodes = z.shape[0]
    table_flat = embed_table.reshape(-1)
    pad = (-table_flat.shape[0]) % 128
    table_pad = jnp.pad(table_flat, (0, pad))

    dense_blk_edges = 512 * 128
    n_pad = -(-n_edges // dense_blk_edges) * dense_blk_edges
    extra = n_pad - n_edges

    w = _sc_gather_w(z.astype(jnp.int32), idx_i.astype(jnp.int32),
                     idx_j.astype(jnp.int32), table_pad, n_nodes, n_edges,
                     n_pad)

    rows = n_pad // 128
    brow = 512
    shp2 = (rows, 128)
    rx = jnp.pad(r_ij[:, 0], (0, extra)).reshape(shp2)
    ry = jnp.pad(r_ij[:, 1], (0, extra)).reshape(shp2)
    rz = jnp.pad(r_ij[:, 2], (0, extra)).reshape(shp2)
    w2 = w.reshape(shp2)
    dense_spec = pl.BlockSpec((brow, 128), lambda i: (i, 0))
    per_edge = pl.pallas_call(
        _tc_dense_body,
        grid=(rows // brow,),
        in_specs=[dense_spec] * 4,
        out_specs=[dense_spec] * 11,
        out_shape=[jax.ShapeDtypeStruct(shp2, jnp.float32)] * 11,
        compiler_params=pltpu.CompilerParams(
            dimension_semantics=("arbitrary",),
        ),
    )(rx, ry, rz, w2)

    q = n_edges // 4
    bq = 2000
    per_edge4 = [a.reshape(n_pad // 4, 4) for a in per_edge]
    out_pack = pl.pallas_call(
        _tc_expand_body,
        grid=(q // bq,),
        in_specs=[pl.BlockSpec((bq, 4), lambda i: (i, 0))] * 11,
        out_specs=pl.BlockSpec((bq, 128), lambda i: (i, 0)),
        out_shape=jax.ShapeDtypeStruct((q, 128), jnp.float32),
        compiler_params=pltpu.CompilerParams(
            dimension_semantics=("arbitrary",),
        ),
    )(*per_edge4)
    return out_pack.reshape(n_edges, 32)


# speedup vs baseline: 83.2993x; 14.2256x over previous
import functools

import jax
import jax.numpy as jnp
from jax import lax
from jax.experimental import pallas as pl
from jax.experimental.pallas import tpu as pltpu
from jax.experimental.pallas import tpu_sc as plsc

_N_RBF = 8
_R_CUT = 5.0
_C0 = 0.28209479177387814
_C1 = 0.4886025119029199

_LANES = 16
_SC_CHUNK = 2000


def _sc_gather_w(z, idx_i, idx_j, table_pad, n_nodes, n_edges):
    info = plsc.get_sparse_core_info()
    nc, ns = info.num_cores, info.num_subcores
    nw = nc * ns
    per_tile = n_edges // nw
    chunk = _SC_CHUNK
    n_chunks = per_tile // chunk
    mesh = plsc.VectorSubcoreMesh(core_axis_name="c", subcore_axis_name="s")

    @functools.partial(
        pl.kernel,
        mesh=mesh,
        out_type=jax.ShapeDtypeStruct((n_edges,), jnp.float32),
        scratch_types=[
            pltpu.VMEM((n_nodes,), jnp.int32),
            pltpu.VMEM((table_pad.shape[0],), jnp.float32),
            pltpu.VMEM((chunk,), jnp.int32),
            pltpu.VMEM((chunk,), jnp.int32),
            pltpu.VMEM((chunk,), jnp.float32),
        ],
        compiler_params=pltpu.CompilerParams(
            needs_layout_passes=False,
            use_tc_tiling_on_sc=False,
        ),
    )
    def body(z_hbm, ii_hbm, jj_hbm, tab_hbm, w_hbm, z_v, tab_v, ii_v, jj_v, w_v):
        wid = lax.axis_index("s") * nc + lax.axis_index("c")
        base = wid * per_tile
        pltpu.sync_copy(z_hbm, z_v)
        pltpu.sync_copy(tab_hbm, tab_v)

        def vec_body(t, _):
            ii = ii_v[pl.ds(t * _LANES, _LANES)]
            jj = jj_v[pl.ds(t * _LANES, _LANES)]
            zi = plsc.load_gather(z_v, [ii])
            zj = plsc.load_gather(z_v, [jj])
            xi = plsc.load_gather(tab_v, [zi])
            xj = plsc.load_gather(tab_v, [zj])
            w_v[pl.ds(t * _LANES, _LANES)] = xi * xj
            return _

        for c in range(n_chunks):
            off = base + c * chunk
            pltpu.sync_copy(ii_hbm.at[pl.ds(off, chunk)], ii_v)
            pltpu.sync_copy(jj_hbm.at[pl.ds(off, chunk)], jj_v)
            lax.fori_loop(0, chunk // _LANES, vec_body, 0)
            pltpu.sync_copy(w_v, w_hbm.at[pl.ds(off, chunk)])

    return body(z, idx_i, idx_j, table_pad)


def _tc_body(rx_ref, ry_ref, rz_ref, w_ref, out_ref):
    rx = rx_ref[...].reshape(1, -1)
    ry = ry_ref[...].reshape(1, -1)
    rz = rz_ref[...].reshape(1, -1)
    w = w_ref[...].reshape(1, -1)
    d2 = rx * rx + ry * ry + rz * rz
    inv_d = lax.rsqrt(d2)
    x = d2 * inv_d * (jnp.pi / _R_CUT)
    s1 = jnp.sin(x)
    c = jnp.cos(x)
    two_c = 2.0 * c
    scale = w * (jnp.sqrt(2.0 / _R_CUT)) * inv_d
    rbf_rows = []
    s_prev, s_cur = jnp.zeros_like(s1), s1
    for _ in range(_N_RBF):
        rbf_rows.append(scale * s_cur)
        s_prev, s_cur = s_cur, two_c * s_cur - s_prev
    rbf = jnp.concatenate(rbf_rows, axis=0)
    c1d = _C1 * inv_d
    out_ref[...] = jnp.concatenate(
        [_C0 * rbf, (c1d * ry) * rbf, (c1d * rz) * rbf, (c1d * rx) * rbf],
        axis=0,
    )


def kernel(z, idx_i, idx_j, r_ij, embed_table):
    n_edges = idx_i.shape[0]
    n_nodes = z.shape[0]
    table_flat = embed_table.reshape(-1)
    pad = (-table_flat.shape[0]) % 128
    table_pad = jnp.pad(table_flat, (0, pad))

    w = _sc_gather_w(z.astype(jnp.int32), idx_i.astype(jnp.int32),
                     idx_j.astype(jnp.int32), table_pad, n_nodes, n_edges)

    bl = 8192
    grid = -(-n_edges // bl)
    in_spec = pl.BlockSpec((bl,), lambda i: (i,))
    out_t = pl.pallas_call(
        _tc_body,
        grid=(grid,),
        in_specs=[in_spec] * 4,
        out_specs=pl.BlockSpec((32, bl), lambda i: (0, i)),
        out_shape=jax.ShapeDtypeStruct((32, n_edges), jnp.float32),
        compiler_params=pltpu.CompilerParams(
            dimension_semantics=("arbitrary",),
        ),
    )(r_ij[:, 0], r_ij[:, 1], r_ij[:, 2], w)
    return out_t.T
